# baseline (speedup 1.0000x reference)
import functools

import jax
import jax.numpy as jnp
from jax import lax
from jax.experimental import pallas as pl
from jax.experimental.pallas import tpu as pltpu
from jax.experimental.pallas import tpu_sc as plsc

VOCAB = 1000000
D = 64
BATCH = 16384
SEQ = 50
NC, NS, L = 2, 16, 16
NW = NC * NS
NCHUNK = BATCH // NW
NBUF = 4
ROUNDS = NCHUNK // NBUF


def _body(table_hbm, idx_hbm, scale_hbm, out_hbm,
          idx_all, scale_v, gbufs, sbufs, gsems, ssems):
    wid = lax.axis_index("s") * NC + lax.axis_index("c")
    base = wid * NCHUNK

    pltpu.sync_copy(idx_hbm.at[wid], idx_all)
    pltpu.sync_copy(scale_hbm, scale_v)
    svec = scale_v[...]

    def fire(c, b):
        pltpu.async_copy(table_hbm.at[idx_all.at[c]], gbufs[b], gsems[b])

    def wait_gather(b):
        pltpu.make_async_copy(table_hbm.at[idx_all.at[0]],
                              gbufs[b], gsems[b]).wait()

    def select_scale(b):
        gbuf, sbuf = gbufs[b], sbufs[b]

        def jstep(j, _):
            for k in range(D // L):
                sl = pl.ds(k * L, L)
                sbuf[j, sl] = gbuf[j, sl] * svec
            return 0
        lax.fori_loop(0, SEQ, jstep, 0)

    def start_store(c, b):
        pltpu.async_copy(sbufs[b], out_hbm.at[base + c], ssems[b])

    def wait_store(b):
        pltpu.make_async_copy(sbufs[b], out_hbm.at[0], ssems[b]).wait()

    for b in range(NBUF):
        fire(b, b)

    def step(t, _):
        c0 = t * NBUF
        for b in range(NBUF):
            wait_gather(b)

            @pl.when(t > 0)
            def _():
                wait_store(b)

            select_scale(b)
            start_store(c0 + b, b)

        @pl.when(t < ROUNDS - 1)
        def _():
            for b in range(NBUF):
                fire(c0 + NBUF + b, b)
        return 0

    lax.fori_loop(0, ROUNDS, step, 0)
    for b in range(NBUF):
        wait_store(b)


@jax.jit
def _embed(table_pad, idx, scale_vec):
    mesh = plsc.VectorSubcoreMesh(core_axis_name="c", subcore_axis_name="s")
    k = pl.kernel(
        _body,
        out_type=jax.ShapeDtypeStruct((BATCH, SEQ, D), jnp.float32),
        mesh=mesh,
        scratch_types=[
            pltpu.VMEM((NCHUNK, SEQ), jnp.int32),
            pltpu.VMEM((L,), jnp.float32),
            [pltpu.VMEM((SEQ, 2 * D), jnp.float32) for _ in range(NBUF)],
            [pltpu.VMEM((SEQ, D), jnp.float32) for _ in range(NBUF)],
            [pltpu.SemaphoreType.DMA for _ in range(NBUF)],
            [pltpu.SemaphoreType.DMA for _ in range(NBUF)],
        ],
        compiler_params=pltpu.CompilerParams(use_tc_tiling_on_sc=True,
                                             needs_layout_passes=False),
    )
    return k(table_pad, idx, scale_vec)


BN = 512
BB = 256


def _tpose_table_body(wt_ref, out_ref):
    out_ref[:, pl.ds(0, D)] = jnp.transpose(wt_ref[...], (1, 0))


@jax.jit
def _table_pad(wt):
    grid = (VOCAB + BN - 1) // BN
    return pl.pallas_call(
        _tpose_table_body,
        grid=(grid,),
        in_specs=[pl.BlockSpec((D, BN), lambda n: (0, n))],
        out_specs=pl.BlockSpec((BN, 2 * D), lambda n: (n, 0)),
        out_shape=jax.ShapeDtypeStruct((VOCAB, 2 * D), jnp.float32),
    )(wt)


def _tpose_out_body(x_ref, out_ref):
    for s in range(SEQ):
        out_ref[s, :, :] = jnp.transpose(x_ref[:, s, :], (1, 0))


@jax.jit
def _tpose_out(x):
    return pl.pallas_call(
        _tpose_out_body,
        grid=(BATCH // BB,),
        in_specs=[pl.BlockSpec((BB, SEQ, D), lambda n: (n, 0, 0))],
        out_specs=pl.BlockSpec((SEQ, D, BB), lambda n: (0, 0, n)),
        out_shape=jax.ShapeDtypeStruct((SEQ, D, BATCH), jnp.float32),
    )(x)


def kernel(token_ids, embed_weight, scale):
    table_pad = _table_pad(embed_weight.T)
    idx = token_ids.reshape(NW, NCHUNK, SEQ).astype(jnp.int32)
    scale_vec = jnp.broadcast_to(scale.astype(jnp.float32), (L,))
    out = _embed(table_pad, idx, scale_vec)
    out_t = _tpose_out(out)
    return jnp.transpose(out_t, (2, 0, 1))


# speedup vs baseline: 1.2648x; 1.2648x over previous
import functools

import jax
import jax.numpy as jnp
from jax import lax
from jax.experimental import pallas as pl
from jax.experimental.pallas import tpu as pltpu
from jax.experimental.pallas import tpu_sc as plsc

VOCAB = 1000000
D = 64
BATCH = 16384
SEQ = 50
NC, NS, L = 2, 16, 16
NW = NC * NS
NCHUNK = BATCH // NW
NBUF = 4
ROUNDS = NCHUNK // NBUF


def _body(table_hbm, idx_hbm, scale_hbm, out_hbm,
          idx_all, scale_v, gbufs, sbufs, gsems, ssems):
    wid = lax.axis_index("s") * NC + lax.axis_index("c")
    base = wid * NCHUNK

    pltpu.sync_copy(idx_hbm.at[wid], idx_all)
    pltpu.sync_copy(scale_hbm, scale_v)
    svec = scale_v[...]

    def fire(c, b):
        pltpu.async_copy(table_hbm.at[idx_all.at[c]], gbufs[b], gsems[b])

    def wait_gather(b):
        pltpu.make_async_copy(table_hbm.at[idx_all.at[0]],
                              gbufs[b], gsems[b]).wait()

    def select_scale(b):
        gbuf, sbuf = gbufs[b], sbufs[b]

        def jstep(j, _):
            for k in range(D // L):
                sl = pl.ds(k * L, L)
                sbuf[j, sl] = gbuf[j, sl] * svec
            return 0
        lax.fori_loop(0, SEQ, jstep, 0)

    def start_store(c, b):
        pltpu.async_copy(sbufs[b], out_hbm.at[base + c], ssems[b])

    def wait_store(b):
        pltpu.make_async_copy(sbufs[b], out_hbm.at[0], ssems[b]).wait()

    for b in range(NBUF):
        fire(b, b)

    def step(t, _):
        c0 = t * NBUF
        for b in range(NBUF):
            wait_gather(b)

            @pl.when(t > 0)
            def _():
                wait_store(b)

            select_scale(b)
            start_store(c0 + b, b)

        @pl.when(t < ROUNDS - 1)
        def _():
            for b in range(NBUF):
                fire(c0 + NBUF + b, b)
        return 0

    lax.fori_loop(0, ROUNDS, step, 0)
    for b in range(NBUF):
        wait_store(b)


@jax.jit
def _embed(table_pad, idx, scale_vec):
    mesh = plsc.VectorSubcoreMesh(core_axis_name="c", subcore_axis_name="s")
    k = pl.kernel(
        _body,
        out_type=jax.ShapeDtypeStruct((BATCH, SEQ, D), jnp.float32),
        mesh=mesh,
        scratch_types=[
            pltpu.VMEM((NCHUNK, SEQ), jnp.int32),
            pltpu.VMEM((L,), jnp.float32),
            [pltpu.VMEM((SEQ, 2 * D), jnp.float32) for _ in range(NBUF)],
            [pltpu.VMEM((SEQ, D), jnp.float32) for _ in range(NBUF)],
            [pltpu.SemaphoreType.DMA for _ in range(NBUF)],
            [pltpu.SemaphoreType.DMA for _ in range(NBUF)],
        ],
        compiler_params=pltpu.CompilerParams(use_tc_tiling_on_sc=True,
                                             needs_layout_passes=False),
    )
    return k(table_pad, idx, scale_vec)


BN = 2000
BB = 256


def _pad_table_body(w_ref, out_ref):
    out_ref[:, pl.ds(0, D)] = w_ref[...]


@jax.jit
def _table_pad(w):
    return pl.pallas_call(
        _pad_table_body,
        grid=(VOCAB // BN,),
        in_specs=[pl.BlockSpec((BN, D), lambda n: (n, 0))],
        out_specs=pl.BlockSpec((BN, 2 * D), lambda n: (n, 0)),
        out_shape=jax.ShapeDtypeStruct((VOCAB, 2 * D), jnp.float32),
    )(w)


def _tpose_out_body(x_ref, out_ref):
    for s in range(SEQ):
        out_ref[s, :, :] = jnp.transpose(x_ref[:, s, :], (1, 0))


@jax.jit
def _tpose_out(x):
    return pl.pallas_call(
        _tpose_out_body,
        grid=(BATCH // BB,),
        in_specs=[pl.BlockSpec((BB, SEQ, D), lambda n: (n, 0, 0))],
        out_specs=pl.BlockSpec((SEQ, D, BB), lambda n: (0, 0, n)),
        out_shape=jax.ShapeDtypeStruct((SEQ, D, BATCH), jnp.float32),
    )(x)


def kernel(token_ids, embed_weight, scale):
    table_pad = _table_pad(embed_weight)
    idx = token_ids.reshape(NW, NCHUNK, SEQ).astype(jnp.int32)
    scale_vec = jnp.broadcast_to(scale.astype(jnp.float32), (L,))
    out = _embed(table_pad, idx, scale_vec)
    out_t = _tpose_out(out)
    return jnp.transpose(out_t, (2, 0, 1))


# speedup vs baseline: 1.6036x; 1.2679x over previous
import functools

import jax
import jax.numpy as jnp
from jax import lax
from jax.experimental import pallas as pl
from jax.experimental.pallas import tpu as pltpu
from jax.experimental.pallas import tpu_sc as plsc

VOCAB = 1000000
D = 64
BATCH = 16384
SEQ = 50
NC, NS, L = 2, 16, 16
NW = NC * NS
NCHUNK = BATCH // NW
NBUF = 4
ROUNDS = NCHUNK // NBUF


def _body(table_hbm, idx_hbm, scale_hbm, out_hbm,
          idx_all, scale_v, gbufs, sbufs, gsems, ssems):
    wid = lax.axis_index("s") * NC + lax.axis_index("c")
    base = wid * NCHUNK

    pltpu.sync_copy(idx_hbm.at[wid], idx_all)
    pltpu.sync_copy(scale_hbm, scale_v)
    svec = scale_v[...]

    def fire(c, b):
        pltpu.async_copy(table_hbm.at[idx_all.at[c]], gbufs[b], gsems[b])

    def wait_gather(b):
        pltpu.make_async_copy(table_hbm.at[idx_all.at[0]],
                              gbufs[b], gsems[b]).wait()

    def select_scale(b):
        gbuf, sbuf = gbufs[b], sbufs[b]

        def jstep(j, _):
            for k in range(D // L):
                sl = pl.ds(k * L, L)
                sbuf[j, sl] = gbuf[j, sl] * svec
            return 0
        lax.fori_loop(0, SEQ, jstep, 0)

    def start_store(c, b):
        pltpu.async_copy(sbufs[b], out_hbm.at[base + c], ssems[b])

    def wait_store(b):
        pltpu.make_async_copy(sbufs[b], out_hbm.at[0], ssems[b]).wait()

    for b in range(NBUF):
        fire(b, b)

    def step(t, _):
        c0 = t * NBUF
        for b in range(NBUF):
            wait_gather(b)

            @pl.when(t > 0)
            def _():
                wait_store(b)

            select_scale(b)
            start_store(c0 + b, b)

        @pl.when(t < ROUNDS - 1)
        def _():
            for b in range(NBUF):
                fire(c0 + NBUF + b, b)
        return 0

    lax.fori_loop(0, ROUNDS, step, 0)
    for b in range(NBUF):
        wait_store(b)


@jax.jit
def _embed(table_pad, idx, scale_vec):
    mesh = plsc.VectorSubcoreMesh(core_axis_name="c", subcore_axis_name="s")
    k = pl.kernel(
        _body,
        out_type=jax.ShapeDtypeStruct((BATCH, SEQ, D), jnp.float32),
        mesh=mesh,
        scratch_types=[
            pltpu.VMEM((NCHUNK, SEQ), jnp.int32),
            pltpu.VMEM((L,), jnp.float32),
            [pltpu.VMEM((SEQ, 2 * D), jnp.float32) for _ in range(NBUF)],
            [pltpu.VMEM((SEQ, D), jnp.float32) for _ in range(NBUF)],
            [pltpu.SemaphoreType.DMA for _ in range(NBUF)],
            [pltpu.SemaphoreType.DMA for _ in range(NBUF)],
        ],
        compiler_params=pltpu.CompilerParams(use_tc_tiling_on_sc=True,
                                             needs_layout_passes=False),
    )
    return k(table_pad, idx, scale_vec)


BB = 256


def _tpose_out_body(x_ref, out_ref):
    for s in range(SEQ):
        out_ref[s, :, :] = jnp.transpose(x_ref[:, s, :], (1, 0))


@jax.jit
def _tpose_out(x):
    return pl.pallas_call(
        _tpose_out_body,
        grid=(BATCH // BB,),
        in_specs=[pl.BlockSpec((BB, SEQ, D), lambda n: (n, 0, 0))],
        out_specs=pl.BlockSpec((SEQ, D, BB), lambda n: (0, 0, n)),
        out_shape=jax.ShapeDtypeStruct((SEQ, D, BATCH), jnp.float32),
    )(x)


def kernel(token_ids, embed_weight, scale):
    table_pad = jnp.pad(embed_weight, ((0, 0), (0, D)))
    idx = token_ids.reshape(NW, NCHUNK, SEQ).astype(jnp.int32)
    scale_vec = jnp.broadcast_to(scale.astype(jnp.float32), (L,))
    out = _embed(table_pad, idx, scale_vec)
    out_t = _tpose_out(out)
    return jnp.transpose(out_t, (2, 0, 1))


# speedup vs baseline: 1.6285x; 1.0155x over previous
import functools

import jax
import jax.numpy as jnp
from jax import lax
from jax.experimental import pallas as pl
from jax.experimental.pallas import tpu as pltpu
from jax.experimental.pallas import tpu_sc as plsc

VOCAB = 1000000
D = 64
BATCH = 16384
SEQ = 50
NC, NS, L = 2, 16, 16
NW = NC * NS
NCHUNK = BATCH // NW
NBUF = 4
ROUNDS = NCHUNK // NBUF


def _body(table_hbm, idx_hbm, scale_hbm, out_hbm,
          idx_all, scale_v, gbufs, sbufs, gsems, ssems):
    wid = lax.axis_index("s") * NC + lax.axis_index("c")
    base = wid * NCHUNK

    pltpu.sync_copy(idx_hbm.at[wid], idx_all)
    pltpu.sync_copy(scale_hbm, scale_v)
    svec = scale_v[...]

    def fire(c, b):
        pltpu.async_copy(table_hbm.at[idx_all.at[c]], gbufs[b], gsems[b])

    def wait_gather(b):
        pltpu.make_async_copy(table_hbm.at[idx_all.at[0]],
                              gbufs[b], gsems[b]).wait()

    def select_scale(b):
        gbuf, sbuf = gbufs[b], sbufs[b]

        def jstep(j, _):
            for k in range(D // L):
                sl = pl.ds(k * L, L)
                sbuf[j, sl] = gbuf[j, sl] * svec
            return 0
        lax.fori_loop(0, SEQ, jstep, 0)

    def start_store(c, b):
        pltpu.async_copy(sbufs[b], out_hbm.at[base + c], ssems[b])

    def wait_store(b):
        pltpu.make_async_copy(sbufs[b], out_hbm.at[0], ssems[b]).wait()

    for b in range(NBUF):
        fire(b, b)

    def step(t, _):
        c0 = t * NBUF
        for b in range(NBUF):
            wait_gather(b)

            @pl.when(t > 0)
            def _():
                wait_store(b)

            select_scale(b)
            start_store(c0 + b, b)

        @pl.when(t < ROUNDS - 1)
        def _():
            for b in range(NBUF):
                fire(c0 + NBUF + b, b)
        return 0

    lax.fori_loop(0, ROUNDS, step, 0)
    for b in range(NBUF):
        wait_store(b)


@jax.jit
def _embed(table_pad, idx, scale_vec):
    mesh = plsc.VectorSubcoreMesh(core_axis_name="c", subcore_axis_name="s")
    k = pl.kernel(
        _body,
        out_type=jax.ShapeDtypeStruct((BATCH, SEQ, D), jnp.float32),
        mesh=mesh,
        scratch_types=[
            pltpu.VMEM((NCHUNK, SEQ), jnp.int32),
            pltpu.VMEM((L,), jnp.float32),
            [pltpu.VMEM((SEQ, 2 * D), jnp.float32) for _ in range(NBUF)],
            [pltpu.VMEM((SEQ, D), jnp.float32) for _ in range(NBUF)],
            [pltpu.SemaphoreType.DMA for _ in range(NBUF)],
            [pltpu.SemaphoreType.DMA for _ in range(NBUF)],
        ],
        compiler_params=pltpu.CompilerParams(use_tc_tiling_on_sc=True,
                                             needs_layout_passes=False),
    )
    return k(table_pad, idx, scale_vec)


BB = 512


def _tpose_out_body(x_ref, out_ref):
    for s in range(SEQ):
        out_ref[s, :, :] = jnp.transpose(x_ref[:, s, :], (1, 0))


@jax.jit
def _tpose_out(x):
    return pl.pallas_call(
        _tpose_out_body,
        grid=(BATCH // BB,),
        in_specs=[pl.BlockSpec((BB, SEQ, D), lambda n: (n, 0, 0))],
        out_specs=pl.BlockSpec((SEQ, D, BB), lambda n: (0, 0, n)),
        out_shape=jax.ShapeDtypeStruct((SEQ, D, BATCH), jnp.float32),
        compiler_params=pltpu.CompilerParams(vmem_limit_bytes=100 << 20),
    )(x)


def kernel(token_ids, embed_weight, scale):
    table_pad = jnp.pad(embed_weight, ((0, 0), (0, D)))
    idx = token_ids.reshape(NW, NCHUNK, SEQ).astype(jnp.int32)
    scale_vec = jnp.broadcast_to(scale.astype(jnp.float32), (L,))
    out = _embed(table_pad, idx, scale_vec)
    out_t = _tpose_out(out)
    return jnp.transpose(out_t, (2, 0, 1))


# speedup vs baseline: 1.6531x; 1.0151x over previous
import functools

import jax
import jax.numpy as jnp
from jax import lax
from jax.experimental import pallas as pl
from jax.experimental.pallas import tpu as pltpu
from jax.experimental.pallas import tpu_sc as plsc

VOCAB = 1000000
D = 64
BATCH = 16384
SEQ = 50
NC, NS, L = 2, 16, 16
NW = NC * NS
NCHUNK = BATCH // NW
NBUF = 4
ROUNDS = NCHUNK // NBUF


def _body(table_hbm, idx_hbm, scale_hbm, out_hbm,
          idx_all, scale_v, gbufs, sbufs, gsems, ssems):
    wid = lax.axis_index("s") * NC + lax.axis_index("c")
    base = wid * NCHUNK

    pltpu.sync_copy(idx_hbm.at[wid], idx_all)
    pltpu.sync_copy(scale_hbm, scale_v)
    svec = scale_v[...]

    def fire(c, b):
        pltpu.async_copy(table_hbm.at[idx_all.at[c]], gbufs[b], gsems[b])

    def wait_gather(b):
        pltpu.make_async_copy(table_hbm.at[idx_all.at[0]],
                              gbufs[b], gsems[b]).wait()

    def select_scale(b):
        gbuf, sbuf = gbufs[b], sbufs[b]

        def jstep(j, _):
            for k in range(D // L):
                sl = pl.ds(k * L, L)
                sbuf[j, sl] = gbuf[j, sl] * svec
            return 0
        lax.fori_loop(0, SEQ, jstep, 0)

    def start_store(c, b):
        pltpu.async_copy(sbufs[b], out_hbm.at[base + c], ssems[b])

    def wait_store(b):
        pltpu.make_async_copy(sbufs[b], out_hbm.at[0], ssems[b]).wait()

    for b in range(NBUF):
        fire(b, b)

    def step(t, _):
        c0 = t * NBUF
        for b in range(NBUF):
            wait_gather(b)

            @pl.when(t > 0)
            def _():
                wait_store(b)

            select_scale(b)

            @pl.when(t < ROUNDS - 1)
            def _():
                fire(c0 + NBUF + b, b)

            start_store(c0 + b, b)
        return 0

    lax.fori_loop(0, ROUNDS, step, 0)
    for b in range(NBUF):
        wait_store(b)


@jax.jit
def _embed(table_pad, idx, scale_vec):
    mesh = plsc.VectorSubcoreMesh(core_axis_name="c", subcore_axis_name="s")
    k = pl.kernel(
        _body,
        out_type=jax.ShapeDtypeStruct((BATCH, SEQ, D), jnp.float32),
        mesh=mesh,
        scratch_types=[
            pltpu.VMEM((NCHUNK, SEQ), jnp.int32),
            pltpu.VMEM((L,), jnp.float32),
            [pltpu.VMEM((SEQ, 2 * D), jnp.float32) for _ in range(NBUF)],
            [pltpu.VMEM((SEQ, D), jnp.float32) for _ in range(NBUF)],
            [pltpu.SemaphoreType.DMA for _ in range(NBUF)],
            [pltpu.SemaphoreType.DMA for _ in range(NBUF)],
        ],
        compiler_params=pltpu.CompilerParams(use_tc_tiling_on_sc=True,
                                             needs_layout_passes=False),
    )
    return k(table_pad, idx, scale_vec)


BB = 512


def _tpose_out_body(x_ref, out_ref):
    for s in range(SEQ):
        out_ref[s, :, :] = jnp.transpose(x_ref[:, s, :], (1, 0))


@jax.jit
def _tpose_out(x):
    return pl.pallas_call(
        _tpose_out_body,
        grid=(BATCH // BB,),
        in_specs=[pl.BlockSpec((BB, SEQ, D), lambda n: (n, 0, 0))],
        out_specs=pl.BlockSpec((SEQ, D, BB), lambda n: (0, 0, n)),
        out_shape=jax.ShapeDtypeStruct((SEQ, D, BATCH), jnp.float32),
        compiler_params=pltpu.CompilerParams(vmem_limit_bytes=100 << 20),
    )(x)


def kernel(token_ids, embed_weight, scale):
    table_pad = jnp.pad(embed_weight, ((0, 0), (0, D)))
    idx = token_ids.reshape(NW, NCHUNK, SEQ).astype(jnp.int32)
    scale_vec = jnp.broadcast_to(scale.astype(jnp.float32), (L,))
    out = _embed(table_pad, idx, scale_vec)
    out_t = _tpose_out(out)
    return jnp.transpose(out_t, (2, 0, 1))
